# baseline (speedup 1.0000x reference)
import functools

import jax
import jax.numpy as jnp
from jax import lax
from jax.experimental import pallas as pl
from jax.experimental.pallas import tpu as pltpu
from jax.experimental.pallas import tpu_sc as plsc

N = 10000
F_IN = 128
F_HID = 64
F_LAT = 16
F_C = 2 * F_LAT

NC = 2
NS = 16
NW = NC * NS
CH = 128

N_PAD = 10112
RPT = N_PAD // NS

_MESH = plsc.VectorSubcoreMesh(core_axis_name="c", subcore_axis_name="s")


def _zero_fill(buf, rows, width):
    zv = jnp.zeros((16,), jnp.float32)

    def body(r, carry):
        for j in range(width // 16):
            buf[r, pl.ds(j * 16, 16)] = zv
        return carry

    lax.fori_loop(0, rows, body, 0)


NB_MAX = 4


def _core_plan(c0, c1, c, s):
    cnt = jnp.where(c == 0, c0, c1)
    start = jnp.where(c == 0, s * c0, NS * c0 + s * c1)
    return cnt, start


def _make_deg_kernel(c0, c1):
    cm = max(c0, c1)

    @functools.partial(
        pl.kernel,
        mesh=_MESH,
        out_type=jax.ShapeDtypeStruct((NC, N_PAD, 16), jnp.float32),
        compiler_params=pltpu.CompilerParams(use_tc_tiling_on_sc=False),
        scratch_types=[
            pltpu.VMEM((cm, CH), jnp.int32),
            pltpu.VMEM((CH, 16), jnp.float32),
            pltpu.VMEM((RPT, 16), jnp.float32),
            pltpu.VMEM_SHARED((N_PAD, 16), jnp.float32),
            pltpu.SemaphoreType.DMA,
        ],
    )
    def deg_kernel(dst_hbm, out_hbm, idx_v, ones_v, zbuf, acc_sh, sem):
        c = lax.axis_index("c")
        s = lax.axis_index("s")
        cnt, start = _core_plan(c0, c1, c, s)

        one = jnp.ones((16,), jnp.float32)

        def fill_ones(r, carry):
            ones_v[r, :] = one
            return carry

        lax.fori_loop(0, CH, fill_ones, 0)
        _zero_fill(zbuf, RPT, 16)
        pltpu.sync_copy(zbuf, acc_sh.at[pl.ds(s * RPT, RPT)])
        pltpu.sync_copy(dst_hbm.at[pl.ds(start, cm)], idx_v)
        plsc.subcore_barrier()

        def chunk(i, carry):
            pltpu.async_copy(ones_v, acc_sh.at[idx_v.at[i]], sem, add=True)
            return carry

        lax.fori_loop(0, cnt, chunk, 0)

        def drain(i, carry):
            pltpu.make_async_copy(ones_v, acc_sh.at[idx_v.at[0]], sem).wait()
            return carry

        lax.fori_loop(0, cnt, drain, 0)
        plsc.subcore_barrier()
        pltpu.sync_copy(acc_sh.at[pl.ds(s * RPT, RPT)],
                        out_hbm.at[c, pl.ds(s * RPT, RPT)])

    return deg_kernel


def _make_agg_kernel(c0, c1, feat, NB):
    assert c0 % (2 * NB) == 0 and c1 % (2 * NB) == 0
    assert c0 > 0
    cm = max(c0, c1)

    @functools.partial(
        pl.kernel,
        mesh=_MESH,
        out_type=jax.ShapeDtypeStruct((NC, N_PAD, feat), jnp.float32),
        compiler_params=pltpu.CompilerParams(use_tc_tiling_on_sc=False),
        scratch_types=[
            pltpu.VMEM((cm, CH), jnp.int32),
            pltpu.VMEM((cm, CH), jnp.int32),
            pltpu.VMEM((2, NB, CH, feat), jnp.float32),
            pltpu.VMEM((RPT // 4, feat), jnp.float32),
            pltpu.VMEM_SHARED((N_PAD, feat), jnp.float32),
            pltpu.SemaphoreType.DMA,
            pltpu.SemaphoreType.DMA,
            pltpu.SemaphoreType.DMA,
        ],
    )
    def agg_kernel(src_hbm, dst_hbm, t_hbm, out_hbm,
                   sidx_v, didx_v, rows_v, zbuf, acc_sh, semg, semsc0, semsc1):
        c = lax.axis_index("c")
        s = lax.axis_index("s")
        _, start = _core_plan(c0, c1, c, s)
        n_super = jnp.where(c == 0, c0 // (2 * NB), c1 // (2 * NB))
        semsc = (semsc0, semsc1)

        _zero_fill(zbuf, RPT // 4, feat)
        for q in range(4):
            pltpu.sync_copy(
                zbuf, acc_sh.at[pl.ds(s * RPT + q * (RPT // 4), RPT // 4)])
        pltpu.sync_copy(src_hbm.at[pl.ds(start, cm)], sidx_v)
        pltpu.sync_copy(dst_hbm.at[pl.ds(start, cm)], didx_v)
        plsc.subcore_barrier()

        def supergroup(k, carry):
            for half in range(2):
                base = (2 * k + half) * NB

                @pl.when(k >= 1)
                def _():
                    for b in range(NB):
                        pltpu.make_async_copy(
                            rows_v.at[half, b], acc_sh.at[didx_v.at[0]],
                            semsc[half]).wait()

                gds = [
                    pltpu.async_copy(
                        t_hbm.at[sidx_v.at[base + b]], rows_v.at[half, b], semg)
                    for b in range(NB)
                ]
                for d in gds:
                    d.wait()
                for b in range(NB):
                    pltpu.async_copy(
                        rows_v.at[half, b], acc_sh.at[didx_v.at[base + b]],
                        semsc[half], add=True)
            return carry

        lax.fori_loop(0, n_super, supergroup, 0)

        @pl.when(n_super > 0)
        def _():
            for half in range(2):
                for b in range(NB):
                    pltpu.make_async_copy(
                        rows_v.at[half, b], acc_sh.at[didx_v.at[0]],
                        semsc[half]).wait()

        plsc.subcore_barrier()
        pltpu.sync_copy(acc_sh.at[pl.ds(s * RPT, RPT)],
                        out_hbm.at[c, pl.ds(s * RPT, RPT)])

    return agg_kernel


_BM = 1000


def _tca_body(x_ref, w1_ref, degp_ref, t1_ref, dinv_ref):
    y1 = jnp.dot(x_ref[...], w1_ref[...], preferred_element_type=jnp.float32)
    cnt = degp_ref[0] + degp_ref[1]
    dinv = lax.rsqrt(cnt[:, 0:1] + 1.0)
    t1_ref[...] = y1 * dinv
    dinv_ref[...] = dinv


def _tcb_body(a1p_ref, t1_ref, dinv_ref, b1_ref, wc_ref, tc2_ref):
    agg = a1p_ref[0] + a1p_ref[1] + t1_ref[...]
    h1 = jnp.maximum(agg * dinv_ref[...] + b1_ref[...], 0.0)
    yc = jnp.dot(h1, wc_ref[...], preferred_element_type=jnp.float32)
    tc2_ref[...] = yc * dinv_ref[...]


def _tcc_body(a2p_ref, tc2_ref, dinv_ref, bc_ref, gn_ref, z_ref):
    u = (a2p_ref[0] + a2p_ref[1] + tc2_ref[...]) * dinv_ref[...] + bc_ref[...]
    xu = u[:, :F_LAT]
    xs = u[:, F_LAT:]
    z_ref[...] = gn_ref[...] * jnp.exp(xu) + xs


_DM = 200


def _tcd_body(z1_ref, z2_ref, out_ref):
    zz = lax.dot_general(z1_ref[...], z2_ref[...],
                         (((1,), (1,)), ((), ())),
                         preferred_element_type=jnp.float32)
    out_ref[...] = jax.nn.sigmoid(zz)


def kernel(x, edge_index, W1, b1, W2, b2, W3, b3):
    n_edges = edge_index.shape[1]
    real_chunks = -(-n_edges // CH)
    tot_pt = -(-real_chunks // NS)
    tot_pt = -(-tot_pt // (2 * NB_MAX)) * (2 * NB_MAX)
    c0 = tot_pt
    c1 = tot_pt - c0
    t_rows = NS * tot_pt + (max(c0, c1) - min(c0, c1))
    e_pad = t_rows * CH

    ei = edge_index.astype(jnp.int32)
    src = jnp.concatenate(
        [ei[0], jnp.zeros((e_pad - n_edges,), jnp.int32)]).reshape(t_rows, CH)
    dst = jnp.concatenate(
        [ei[1], jnp.full((e_pad - n_edges,), N, jnp.int32)]).reshape(t_rows, CH)

    degp = _make_deg_kernel(c0, c1)(dst)[:, :N, :]

    grid = (N // _BM,)
    t1, dinv = pl.pallas_call(
        _tca_body,
        grid=grid,
        in_specs=[
            pl.BlockSpec((_BM, F_IN), lambda i: (i, 0)),
            pl.BlockSpec((F_IN, F_HID), lambda i: (0, 0)),
            pl.BlockSpec((NC, _BM, 16), lambda i: (0, i, 0)),
        ],
        out_specs=[
            pl.BlockSpec((_BM, F_HID), lambda i: (i, 0)),
            pl.BlockSpec((_BM, 1), lambda i: (i, 0)),
        ],
        out_shape=[
            jax.ShapeDtypeStruct((N, F_HID), jnp.float32),
            jax.ShapeDtypeStruct((N, 1), jnp.float32),
        ],
    )(x, W1, degp)

    a1p = _make_agg_kernel(c0, c1, F_HID, 2)(src, dst, t1)[:, :N, :]

    Wc = jnp.concatenate([W2, W3], axis=1)
    bc = jnp.concatenate([b2, b3]).reshape(1, F_C)
    tc2 = pl.pallas_call(
        _tcb_body,
        grid=grid,
        in_specs=[
            pl.BlockSpec((NC, _BM, F_HID), lambda i: (0, i, 0)),
            pl.BlockSpec((_BM, F_HID), lambda i: (i, 0)),
            pl.BlockSpec((_BM, 1), lambda i: (i, 0)),
            pl.BlockSpec((1, F_HID), lambda i: (0, 0)),
            pl.BlockSpec((F_HID, F_C), lambda i: (0, 0)),
        ],
        out_specs=pl.BlockSpec((_BM, F_C), lambda i: (i, 0)),
        out_shape=jax.ShapeDtypeStruct((N, F_C), jnp.float32),
    )(a1p, t1, dinv, b1.reshape(1, F_HID), Wc)

    a2p = _make_agg_kernel(c0, c1, F_C, 4)(src, dst, tc2)[:, :N, :]

    gnoise = jax.random.normal(jax.random.key(42), (N, F_LAT), dtype=jnp.float32)
    z = pl.pallas_call(
        _tcc_body,
        grid=grid,
        in_specs=[
            pl.BlockSpec((NC, _BM, F_C), lambda i: (0, i, 0)),
            pl.BlockSpec((_BM, F_C), lambda i: (i, 0)),
            pl.BlockSpec((_BM, 1), lambda i: (i, 0)),
            pl.BlockSpec((1, F_C), lambda i: (0, 0)),
            pl.BlockSpec((_BM, F_LAT), lambda i: (i, 0)),
        ],
        out_specs=pl.BlockSpec((_BM, F_LAT), lambda i: (i, 0)),
        out_shape=jax.ShapeDtypeStruct((N, F_LAT), jnp.float32),
    )(a2p, tc2, dinv, bc, gnoise)

    adj = pl.pallas_call(
        _tcd_body,
        grid=(N // _DM,),
        in_specs=[
            pl.BlockSpec((_DM, F_LAT), lambda i: (i, 0)),
            pl.BlockSpec((N, F_LAT), lambda i: (0, 0)),
        ],
        out_specs=pl.BlockSpec((_DM, N), lambda i: (i, 0)),
        out_shape=jax.ShapeDtypeStruct((N, N), jnp.float32),
    )(z, z)
    return adj


# speedup vs baseline: 1.2043x; 1.2043x over previous
import functools

import jax
import jax.numpy as jnp
from jax import lax
from jax.experimental import pallas as pl
from jax.experimental.pallas import tpu as pltpu
from jax.experimental.pallas import tpu_sc as plsc

N = 10000
F_IN = 128
F_HID = 64
F_LAT = 16
F_C = 2 * F_LAT

NC = 2
NS = 16
NW = NC * NS
CH = 128

N_PAD = 10112
RPT = N_PAD // NS

_MESH = plsc.VectorSubcoreMesh(core_axis_name="c", subcore_axis_name="s")


def _zero_fill(buf, rows, width):
    zv = jnp.zeros((16,), jnp.float32)

    def body(r, carry):
        for j in range(width // 16):
            buf[r, pl.ds(j * 16, 16)] = zv
        return carry

    lax.fori_loop(0, rows, body, 0)


NB_MAX = 4


def _core_plan(c0, c1, c, s):
    cnt = jnp.where(c == 0, c0, c1)
    start = jnp.where(c == 0, s * c0, NS * c0 + s * c1)
    return cnt, start


def _make_deg_kernel(c0, c1):
    cm = max(c0, c1)

    @functools.partial(
        pl.kernel,
        mesh=_MESH,
        out_type=jax.ShapeDtypeStruct((NC, N_PAD, 16), jnp.float32),
        compiler_params=pltpu.CompilerParams(use_tc_tiling_on_sc=False),
        scratch_types=[
            pltpu.VMEM((cm, CH), jnp.int32),
            pltpu.VMEM((CH, 16), jnp.float32),
            pltpu.VMEM((RPT, 16), jnp.float32),
            pltpu.VMEM_SHARED((N_PAD, 16), jnp.float32),
            pltpu.SemaphoreType.DMA,
        ],
    )
    def deg_kernel(dst_hbm, out_hbm, idx_v, ones_v, zbuf, acc_sh, sem):
        c = lax.axis_index("c")
        s = lax.axis_index("s")
        cnt, start = _core_plan(c0, c1, c, s)

        one = jnp.ones((16,), jnp.float32)

        def fill_ones(r, carry):
            ones_v[r, :] = one
            return carry

        lax.fori_loop(0, CH, fill_ones, 0)
        _zero_fill(zbuf, RPT, 16)
        pltpu.sync_copy(zbuf, acc_sh.at[pl.ds(s * RPT, RPT)])
        pltpu.sync_copy(dst_hbm.at[pl.ds(start, cm)], idx_v)
        plsc.subcore_barrier()

        def chunk(i, carry):
            pltpu.async_copy(ones_v, acc_sh.at[idx_v.at[i]], sem, add=True)
            return carry

        lax.fori_loop(0, cnt, chunk, 0)

        def drain(i, carry):
            pltpu.make_async_copy(ones_v, acc_sh.at[idx_v.at[0]], sem).wait()
            return carry

        lax.fori_loop(0, cnt, drain, 0)
        plsc.subcore_barrier()
        pltpu.sync_copy(acc_sh.at[pl.ds(s * RPT, RPT)],
                        out_hbm.at[c, pl.ds(s * RPT, RPT)])

    return deg_kernel


def _make_agg_kernel(c0, c1, feat, NB):
    assert c0 % (2 * NB) == 0 and c1 % (2 * NB) == 0
    assert c0 > 0
    cm = max(c0, c1)

    @functools.partial(
        pl.kernel,
        mesh=_MESH,
        out_type=jax.ShapeDtypeStruct((NC, N_PAD, feat), jnp.float32),
        compiler_params=pltpu.CompilerParams(use_tc_tiling_on_sc=False),
        scratch_types=[
            pltpu.VMEM((cm, CH), jnp.int32),
            pltpu.VMEM((cm, CH), jnp.int32),
            pltpu.VMEM((2, NB, CH, feat), jnp.float32),
            pltpu.VMEM((RPT // 4, feat), jnp.float32),
            pltpu.VMEM_SHARED((N_PAD, feat), jnp.float32),
            pltpu.SemaphoreType.DMA,
            pltpu.SemaphoreType.DMA,
            pltpu.SemaphoreType.DMA,
        ],
    )
    def agg_kernel(src_hbm, dst_hbm, t_hbm, out_hbm,
                   sidx_v, didx_v, rows_v, zbuf, acc_sh, semg, semsc0, semsc1):
        c = lax.axis_index("c")
        s = lax.axis_index("s")
        _, start = _core_plan(c0, c1, c, s)
        n_super = c0 // (2 * NB)
        semsc = (semsc0, semsc1)

        _zero_fill(zbuf, RPT // 4, feat)
        for q in range(4):
            pltpu.sync_copy(
                zbuf, acc_sh.at[pl.ds(s * RPT + q * (RPT // 4), RPT // 4)])
        pltpu.sync_copy(src_hbm.at[pl.ds(start, cm)], sidx_v)
        pltpu.sync_copy(dst_hbm.at[pl.ds(start, cm)], didx_v)
        plsc.subcore_barrier()

        @pl.when(c == 0)
        def _():
            def supergroup(k, carry):
                for half in range(2):
                    base = (2 * k + half) * NB

                    @pl.when(k >= 1)
                    def _():
                        for b in range(NB):
                            pltpu.make_async_copy(
                                rows_v.at[half, b], acc_sh.at[didx_v.at[0]],
                                semsc[half]).wait()

                    gds = [
                        pltpu.async_copy(
                            t_hbm.at[sidx_v.at[base + b]], rows_v.at[half, b],
                            semg)
                        for b in range(NB)
                    ]
                    for d in gds:
                        d.wait()
                    for b in range(NB):
                        pltpu.async_copy(
                            rows_v.at[half, b], acc_sh.at[didx_v.at[base + b]],
                            semsc[half], add=True)
                return carry

            lax.fori_loop(0, n_super, supergroup, 0)
            for half in range(2):
                for b in range(NB):
                    pltpu.make_async_copy(
                        rows_v.at[half, b], acc_sh.at[didx_v.at[0]],
                        semsc[half]).wait()

        @pl.when(c == 1)
        def _():
            def chunk(i, carry):
                pltpu.async_copy(
                    t_hbm.at[sidx_v.at[i]], rows_v.at[0, 0], semg).wait()
                pltpu.sync_copy(
                    rows_v.at[0, 0], acc_sh.at[didx_v.at[i]], add=True)
                return carry

            lax.fori_loop(0, c1, chunk, 0)

        plsc.subcore_barrier()
        pltpu.sync_copy(acc_sh.at[pl.ds(s * RPT, RPT)],
                        out_hbm.at[c, pl.ds(s * RPT, RPT)])

    return agg_kernel


_BM = 1000


def _tca_body(x_ref, w1_ref, degp_ref, t1_ref, dinv_ref):
    y1 = jnp.dot(x_ref[...], w1_ref[...], preferred_element_type=jnp.float32)
    cnt = degp_ref[0] + degp_ref[1]
    dinv = lax.rsqrt(cnt[:, 0:1] + 1.0)
    t1_ref[...] = y1 * dinv
    dinv_ref[...] = dinv


def _tcb_body(a1p_ref, t1_ref, dinv_ref, b1_ref, wc_ref, tc2_ref):
    agg = a1p_ref[0] + a1p_ref[1] + t1_ref[...]
    h1 = jnp.maximum(agg * dinv_ref[...] + b1_ref[...], 0.0)
    yc = jnp.dot(h1, wc_ref[...], preferred_element_type=jnp.float32)
    tc2_ref[...] = yc * dinv_ref[...]


def _tcc_body(a2p_ref, tc2_ref, dinv_ref, bc_ref, gn_ref, z_ref):
    u = (a2p_ref[0] + a2p_ref[1] + tc2_ref[...]) * dinv_ref[...] + bc_ref[...]
    xu = u[:, :F_LAT]
    xs = u[:, F_LAT:]
    z_ref[...] = gn_ref[...] * jnp.exp(xu) + xs


_DM = 200


def _tcd_body(z1_ref, z2_ref, out_ref):
    zz = lax.dot_general(z1_ref[...], z2_ref[...],
                         (((1,), (1,)), ((), ())),
                         preferred_element_type=jnp.float32)
    out_ref[...] = jax.nn.sigmoid(zz)


def kernel(x, edge_index, W1, b1, W2, b2, W3, b3):
    n_edges = edge_index.shape[1]
    real_chunks = -(-n_edges // CH)
    tot_pt = -(-real_chunks // NS)
    tot_pt = -(-tot_pt // (2 * NB_MAX)) * (2 * NB_MAX)
    c0 = (tot_pt * 3 // 4) // (2 * NB_MAX) * (2 * NB_MAX)
    c1 = tot_pt - c0
    t_rows = NS * tot_pt + (max(c0, c1) - min(c0, c1))
    e_pad = t_rows * CH

    ei = edge_index.astype(jnp.int32)
    src = jnp.concatenate(
        [ei[0], jnp.zeros((e_pad - n_edges,), jnp.int32)]).reshape(t_rows, CH)
    dst = jnp.concatenate(
        [ei[1], jnp.full((e_pad - n_edges,), N, jnp.int32)]).reshape(t_rows, CH)

    degp = _make_deg_kernel(c0, c1)(dst)[:, :N, :]

    grid = (N // _BM,)
    t1, dinv = pl.pallas_call(
        _tca_body,
        grid=grid,
        in_specs=[
            pl.BlockSpec((_BM, F_IN), lambda i: (i, 0)),
            pl.BlockSpec((F_IN, F_HID), lambda i: (0, 0)),
            pl.BlockSpec((NC, _BM, 16), lambda i: (0, i, 0)),
        ],
        out_specs=[
            pl.BlockSpec((_BM, F_HID), lambda i: (i, 0)),
            pl.BlockSpec((_BM, 1), lambda i: (i, 0)),
        ],
        out_shape=[
            jax.ShapeDtypeStruct((N, F_HID), jnp.float32),
            jax.ShapeDtypeStruct((N, 1), jnp.float32),
        ],
    )(x, W1, degp)

    a1p = _make_agg_kernel(c0, c1, F_HID, 2)(src, dst, t1)[:, :N, :]

    Wc = jnp.concatenate([W2, W3], axis=1)
    bc = jnp.concatenate([b2, b3]).reshape(1, F_C)
    tc2 = pl.pallas_call(
        _tcb_body,
        grid=grid,
        in_specs=[
            pl.BlockSpec((NC, _BM, F_HID), lambda i: (0, i, 0)),
            pl.BlockSpec((_BM, F_HID), lambda i: (i, 0)),
            pl.BlockSpec((_BM, 1), lambda i: (i, 0)),
            pl.BlockSpec((1, F_HID), lambda i: (0, 0)),
            pl.BlockSpec((F_HID, F_C), lambda i: (0, 0)),
        ],
        out_specs=pl.BlockSpec((_BM, F_C), lambda i: (i, 0)),
        out_shape=jax.ShapeDtypeStruct((N, F_C), jnp.float32),
    )(a1p, t1, dinv, b1.reshape(1, F_HID), Wc)

    a2p = _make_agg_kernel(c0, c1, F_C, 4)(src, dst, tc2)[:, :N, :]

    gnoise = jax.random.normal(jax.random.key(42), (N, F_LAT), dtype=jnp.float32)
    z = pl.pallas_call(
        _tcc_body,
        grid=grid,
        in_specs=[
            pl.BlockSpec((NC, _BM, F_C), lambda i: (0, i, 0)),
            pl.BlockSpec((_BM, F_C), lambda i: (i, 0)),
            pl.BlockSpec((_BM, 1), lambda i: (i, 0)),
            pl.BlockSpec((1, F_C), lambda i: (0, 0)),
            pl.BlockSpec((_BM, F_LAT), lambda i: (i, 0)),
        ],
        out_specs=pl.BlockSpec((_BM, F_LAT), lambda i: (i, 0)),
        out_shape=jax.ShapeDtypeStruct((N, F_LAT), jnp.float32),
    )(a2p, tc2, dinv, bc, gnoise)

    adj = pl.pallas_call(
        _tcd_body,
        grid=(N // _DM,),
        in_specs=[
            pl.BlockSpec((_DM, F_LAT), lambda i: (i, 0)),
            pl.BlockSpec((N, F_LAT), lambda i: (0, 0)),
        ],
        out_specs=pl.BlockSpec((_DM, N), lambda i: (i, 0)),
        out_shape=jax.ShapeDtypeStruct((N, N), jnp.float32),
    )(z, z)
    return adj


# speedup vs baseline: 1.8518x; 1.5377x over previous
import functools

import jax
import jax.numpy as jnp
from jax import lax
from jax.experimental import pallas as pl
from jax.experimental.pallas import tpu as pltpu
from jax.experimental.pallas import tpu_sc as plsc

N = 10000
F_IN = 128
F_HID = 64
F_LAT = 16
F_C = 2 * F_LAT

NC = 2
NS = 16
NW = NC * NS
CH = 128

N_PAD = 10112
RPT = N_PAD // NS

_MESH = plsc.VectorSubcoreMesh(core_axis_name="c", subcore_axis_name="s")


def _zero_fill(buf, rows, width):
    zv = jnp.zeros((16,), jnp.float32)

    def body(r, carry):
        for j in range(width // 16):
            buf[r, pl.ds(j * 16, 16)] = zv
        return carry

    lax.fori_loop(0, rows, body, 0)


NB_MAX = 4


def _core_plan(c0, c1, c, s):
    cnt = jnp.where(c == 0, c0, c1)
    start = jnp.where(c == 0, s * c0, NS * c0 + s * c1)
    return cnt, start


def _make_deg_kernel(c0, c1):
    cm = max(c0, c1)

    @functools.partial(
        pl.kernel,
        mesh=_MESH,
        out_type=jax.ShapeDtypeStruct((NC, N_PAD, 16), jnp.float32),
        compiler_params=pltpu.CompilerParams(use_tc_tiling_on_sc=False),
        scratch_types=[
            pltpu.VMEM((cm, CH), jnp.int32),
            pltpu.VMEM((CH, 16), jnp.float32),
            pltpu.VMEM((RPT, 16), jnp.float32),
            pltpu.VMEM_SHARED((N_PAD, 16), jnp.float32),
            pltpu.SemaphoreType.DMA,
        ],
    )
    def deg_kernel(dst_hbm, out_hbm, idx_v, ones_v, zbuf, acc_sh, sem):
        c = lax.axis_index("c")
        s = lax.axis_index("s")
        cnt, start = _core_plan(c0, c1, c, s)

        one = jnp.ones((16,), jnp.float32)

        def fill_ones(r, carry):
            ones_v[r, :] = one
            return carry

        lax.fori_loop(0, CH, fill_ones, 0)
        _zero_fill(zbuf, RPT, 16)
        pltpu.sync_copy(zbuf, acc_sh.at[pl.ds(s * RPT, RPT)])
        pltpu.sync_copy(dst_hbm.at[pl.ds(start, cm)], idx_v)
        plsc.subcore_barrier()

        def chunk(i, carry):
            pltpu.async_copy(ones_v, acc_sh.at[idx_v.at[i]], sem, add=True)
            return carry

        lax.fori_loop(0, cnt, chunk, 0)

        def drain(i, carry):
            pltpu.make_async_copy(ones_v, acc_sh.at[idx_v.at[0]], sem).wait()
            return carry

        lax.fori_loop(0, cnt, drain, 0)
        plsc.subcore_barrier()
        pltpu.sync_copy(acc_sh.at[pl.ds(s * RPT, RPT)],
                        out_hbm.at[c, pl.ds(s * RPT, RPT)])

    return deg_kernel


def _make_agg_kernel(c0, c1, feat, NB):
    assert c0 % (2 * NB) == 0 and c1 % (2 * NB) == 0
    assert c0 == c1
    cm = max(c0, c1)
    n_super = c0 // (2 * NB)

    @functools.partial(
        pl.kernel,
        mesh=_MESH,
        out_type=jax.ShapeDtypeStruct((NC, N_PAD, feat), jnp.float32),
        compiler_params=pltpu.CompilerParams(use_tc_tiling_on_sc=False),
        scratch_types=[
            pltpu.VMEM((cm, CH), jnp.int32),
            pltpu.VMEM((cm, CH), jnp.int32),
            pltpu.VMEM((2, NB, CH, feat), jnp.float32),
            pltpu.VMEM((RPT // 4, feat), jnp.float32),
            pltpu.VMEM_SHARED((N_PAD, feat), jnp.float32),
            pltpu.VMEM_SHARED((N_PAD, feat), jnp.float32),
            pltpu.SemaphoreType.DMA,
            pltpu.SemaphoreType.DMA,
            pltpu.SemaphoreType.DMA,
        ],
    )
    def agg_kernel(src_hbm, dst_hbm, t_hbm, out_hbm,
                   sidx_v, didx_v, rows_v, zbuf, acc_sh, t_sh,
                   semg, semsc0, semsc1):
        c = lax.axis_index("c")
        s = lax.axis_index("s")
        _, start = _core_plan(c0, c1, c, s)
        semsc = (semsc0, semsc1)

        _zero_fill(zbuf, RPT // 4, feat)
        for q in range(4):
            pltpu.sync_copy(
                zbuf, acc_sh.at[pl.ds(s * RPT + q * (RPT // 4), RPT // 4)])
        pltpu.sync_copy(t_hbm.at[pl.ds(s * RPT, RPT)],
                        t_sh.at[pl.ds(s * RPT, RPT)])
        pltpu.sync_copy(src_hbm.at[pl.ds(start, cm)], sidx_v)
        pltpu.sync_copy(dst_hbm.at[pl.ds(start, cm)], didx_v)
        plsc.subcore_barrier()

        def supergroup(k, carry):
            for half in range(2):
                base = (2 * k + half) * NB

                @pl.when(k >= 1)
                def _():
                    for b in range(NB):
                        pltpu.make_async_copy(
                            rows_v.at[half, b], acc_sh.at[didx_v.at[0]],
                            semsc[half]).wait()

                gds = [
                    pltpu.async_copy(
                        t_sh.at[sidx_v.at[base + b]], rows_v.at[half, b], semg)
                    for b in range(NB)
                ]
                for d in gds:
                    d.wait()
                for b in range(NB):
                    pltpu.async_copy(
                        rows_v.at[half, b], acc_sh.at[didx_v.at[base + b]],
                        semsc[half], add=True)
            return carry

        lax.fori_loop(0, n_super, supergroup, 0)
        for half in range(2):
            for b in range(NB):
                pltpu.make_async_copy(
                    rows_v.at[half, b], acc_sh.at[didx_v.at[0]],
                    semsc[half]).wait()
        plsc.subcore_barrier()
        pltpu.sync_copy(acc_sh.at[pl.ds(s * RPT, RPT)],
                        out_hbm.at[c, pl.ds(s * RPT, RPT)])

    return agg_kernel


_BM = 1000


def _tca_body(x_ref, w1_ref, degp_ref, t1_ref, dinv_ref):
    y1 = jnp.dot(x_ref[...], w1_ref[...], preferred_element_type=jnp.float32)
    cnt = degp_ref[0] + degp_ref[1]
    dinv = lax.rsqrt(cnt[:, 0:1] + 1.0)
    t1_ref[...] = y1 * dinv
    dinv_ref[...] = dinv


def _tcb_body(a1p_ref, t1_ref, dinv_ref, b1_ref, wc_ref, tc2_ref):
    agg = a1p_ref[0] + a1p_ref[1] + t1_ref[...]
    h1 = jnp.maximum(agg * dinv_ref[...] + b1_ref[...], 0.0)
    yc = jnp.dot(h1, wc_ref[...], preferred_element_type=jnp.float32)
    tc2_ref[...] = yc * dinv_ref[...]


def _tcc_body(a2p_ref, tc2_ref, dinv_ref, bc_ref, gn_ref, z_ref):
    u = (a2p_ref[0] + a2p_ref[1] + tc2_ref[...]) * dinv_ref[...] + bc_ref[...]
    xu = u[:, :F_LAT]
    xs = u[:, F_LAT:]
    z_ref[...] = gn_ref[...] * jnp.exp(xu) + xs


_DM = 200


def _tcd_body(z1_ref, z2_ref, out_ref):
    zz = lax.dot_general(z1_ref[...], z2_ref[...],
                         (((1,), (1,)), ((), ())),
                         preferred_element_type=jnp.float32)
    out_ref[...] = jax.nn.sigmoid(zz)


def kernel(x, edge_index, W1, b1, W2, b2, W3, b3):
    n_edges = edge_index.shape[1]
    real_chunks = -(-n_edges // CH)
    tot_pt = -(-real_chunks // NS)
    tot_pt = -(-tot_pt // (2 * NB_MAX)) * (2 * NB_MAX)
    c0 = tot_pt // 2
    c1 = tot_pt - c0
    t_rows = NS * tot_pt + (max(c0, c1) - min(c0, c1))
    e_pad = t_rows * CH

    ei = edge_index.astype(jnp.int32)
    src = jnp.concatenate(
        [ei[0], jnp.zeros((e_pad - n_edges,), jnp.int32)]).reshape(t_rows, CH)
    dst = jnp.concatenate(
        [ei[1], jnp.full((e_pad - n_edges,), N, jnp.int32)]).reshape(t_rows, CH)

    degp = _make_deg_kernel(c0, c1)(dst)

    xp = jnp.concatenate(
        [x, jnp.zeros((N_PAD - N, F_IN), jnp.float32)], axis=0)
    grid_p = (N_PAD // RPT,)
    t1, dinv = pl.pallas_call(
        _tca_body,
        grid=grid_p,
        in_specs=[
            pl.BlockSpec((RPT, F_IN), lambda i: (i, 0)),
            pl.BlockSpec((F_IN, F_HID), lambda i: (0, 0)),
            pl.BlockSpec((NC, RPT, 16), lambda i: (0, i, 0)),
        ],
        out_specs=[
            pl.BlockSpec((RPT, F_HID), lambda i: (i, 0)),
            pl.BlockSpec((RPT, 1), lambda i: (i, 0)),
        ],
        out_shape=[
            jax.ShapeDtypeStruct((N_PAD, F_HID), jnp.float32),
            jax.ShapeDtypeStruct((N_PAD, 1), jnp.float32),
        ],
    )(xp, W1, degp)

    a1p = _make_agg_kernel(c0, c1, F_HID, 1)(src, dst, t1)

    Wc = jnp.concatenate([W2, W3], axis=1)
    bc = jnp.concatenate([b2, b3]).reshape(1, F_C)
    tc2 = pl.pallas_call(
        _tcb_body,
        grid=grid_p,
        in_specs=[
            pl.BlockSpec((NC, RPT, F_HID), lambda i: (0, i, 0)),
            pl.BlockSpec((RPT, F_HID), lambda i: (i, 0)),
            pl.BlockSpec((RPT, 1), lambda i: (i, 0)),
            pl.BlockSpec((1, F_HID), lambda i: (0, 0)),
            pl.BlockSpec((F_HID, F_C), lambda i: (0, 0)),
        ],
        out_specs=pl.BlockSpec((RPT, F_C), lambda i: (i, 0)),
        out_shape=jax.ShapeDtypeStruct((N_PAD, F_C), jnp.float32),
    )(a1p, t1, dinv, b1.reshape(1, F_HID), Wc)

    a2p = _make_agg_kernel(c0, c1, F_C, 2)(src, dst, tc2)

    gnoise = jax.random.normal(jax.random.key(42), (N, F_LAT), dtype=jnp.float32)
    grid = (N // _BM,)
    z = pl.pallas_call(
        _tcc_body,
        grid=grid,
        in_specs=[
            pl.BlockSpec((NC, _BM, F_C), lambda i: (0, i, 0)),
            pl.BlockSpec((_BM, F_C), lambda i: (i, 0)),
            pl.BlockSpec((_BM, 1), lambda i: (i, 0)),
            pl.BlockSpec((1, F_C), lambda i: (0, 0)),
            pl.BlockSpec((_BM, F_LAT), lambda i: (i, 0)),
        ],
        out_specs=pl.BlockSpec((_BM, F_LAT), lambda i: (i, 0)),
        out_shape=jax.ShapeDtypeStruct((N, F_LAT), jnp.float32),
    )(a2p, tc2, dinv, bc, gnoise)

    adj = pl.pallas_call(
        _tcd_body,
        grid=(N // _DM,),
        in_specs=[
            pl.BlockSpec((_DM, F_LAT), lambda i: (i, 0)),
            pl.BlockSpec((N, F_LAT), lambda i: (0, 0)),
        ],
        out_specs=pl.BlockSpec((_DM, N), lambda i: (i, 0)),
        out_shape=jax.ShapeDtypeStruct((N, N), jnp.float32),
    )(z, z)
    return adj


# speedup vs baseline: 2.0087x; 1.0847x over previous
import functools

import jax
import jax.numpy as jnp
from jax import lax
from jax.experimental import pallas as pl
from jax.experimental.pallas import tpu as pltpu
from jax.experimental.pallas import tpu_sc as plsc

N = 10000
F_IN = 128
F_HID = 64
F_LAT = 16
F_C = 2 * F_LAT

NC = 2
NS = 16
NW = NC * NS
CH = 128

N_PAD = 10112
RPT = N_PAD // NS

_MESH = plsc.VectorSubcoreMesh(core_axis_name="c", subcore_axis_name="s")


def _zero_fill(buf, rows, width):
    zv = jnp.zeros((16,), jnp.float32)

    def body(r, carry):
        for j in range(width // 16):
            buf[r, pl.ds(j * 16, 16)] = zv
        return carry

    lax.fori_loop(0, rows, body, 0)


NB_MAX = 4


def _tile_plan(tot, c, s):
    wid = (s * NC + c).astype(jnp.int32)
    base = tot // NW
    extra = tot % NW
    cnt = jnp.where(wid < extra, base + 1, base)
    start = wid * base + jnp.minimum(wid, extra)
    return cnt, start


def _make_deg_kernel(tot):
    cm = -(-tot // NW)

    @functools.partial(
        pl.kernel,
        mesh=_MESH,
        out_type=jax.ShapeDtypeStruct((NC, N_PAD, 16), jnp.float32),
        compiler_params=pltpu.CompilerParams(use_tc_tiling_on_sc=False),
        scratch_types=[
            pltpu.VMEM((cm, CH), jnp.int32),
            pltpu.VMEM((CH, 16), jnp.float32),
            pltpu.VMEM((RPT, 16), jnp.float32),
            pltpu.VMEM_SHARED((N_PAD, 16), jnp.float32),
            pltpu.SemaphoreType.DMA,
        ],
    )
    def deg_kernel(ei_hbm, out_hbm, idx_v, ones_v, zbuf, acc_sh, sem):
        c = lax.axis_index("c")
        s = lax.axis_index("s")
        cnt, start = _tile_plan(tot, c, s)

        one = jnp.ones((16,), jnp.float32)

        def fill_ones(r, carry):
            ones_v[r, :] = one
            return carry

        lax.fori_loop(0, CH, fill_ones, 0)
        _zero_fill(zbuf, RPT, 16)
        pltpu.sync_copy(zbuf, acc_sh.at[pl.ds(s * RPT, RPT)])
        pltpu.sync_copy(ei_hbm.at[1, pl.ds(start, cm)], idx_v)
        plsc.subcore_barrier()

        def chunk(i, carry):
            pltpu.async_copy(ones_v, acc_sh.at[idx_v.at[i]], sem, add=True)
            return carry

        lax.fori_loop(0, cnt, chunk, 0)

        def drain(i, carry):
            pltpu.make_async_copy(ones_v, acc_sh.at[idx_v.at[0]], sem).wait()
            return carry

        lax.fori_loop(0, cnt, drain, 0)
        plsc.subcore_barrier()
        pltpu.sync_copy(acc_sh.at[pl.ds(s * RPT, RPT)],
                        out_hbm.at[c, pl.ds(s * RPT, RPT)])

    return deg_kernel


def _make_agg_kernel(tot, feat, NB):
    cm = -(-tot // NW)

    @functools.partial(
        pl.kernel,
        mesh=_MESH,
        out_type=jax.ShapeDtypeStruct((NC, N_PAD, feat), jnp.float32),
        compiler_params=pltpu.CompilerParams(use_tc_tiling_on_sc=False),
        scratch_types=[
            pltpu.VMEM((cm, CH), jnp.int32),
            pltpu.VMEM((cm, CH), jnp.int32),
            pltpu.VMEM((2, NB, CH, feat), jnp.float32),
            pltpu.VMEM((RPT // 4, feat), jnp.float32),
            pltpu.VMEM_SHARED((N_PAD, feat), jnp.float32),
            pltpu.VMEM_SHARED((N_PAD, feat), jnp.float32),
            pltpu.SemaphoreType.DMA,
            pltpu.SemaphoreType.DMA,
            pltpu.SemaphoreType.DMA,
        ],
    )
    def agg_kernel(ei_hbm, t_hbm, out_hbm,
                   sidx_v, didx_v, rows_v, zbuf, acc_sh, t_sh,
                   semg, semsc0, semsc1):
        c = lax.axis_index("c")
        s = lax.axis_index("s")
        cnt, start = _tile_plan(tot, c, s)
        n_super = cnt // (2 * NB)
        n_tail = cnt - n_super * (2 * NB)
        semsc = (semsc0, semsc1)

        _zero_fill(zbuf, RPT // 4, feat)
        for q in range(4):
            pltpu.sync_copy(
                zbuf, acc_sh.at[pl.ds(s * RPT + q * (RPT // 4), RPT // 4)])
        pltpu.sync_copy(t_hbm.at[pl.ds(s * RPT, RPT)],
                        t_sh.at[pl.ds(s * RPT, RPT)])
        pltpu.sync_copy(ei_hbm.at[0, pl.ds(start, cm)], sidx_v)
        pltpu.sync_copy(ei_hbm.at[1, pl.ds(start, cm)], didx_v)
        plsc.subcore_barrier()

        def supergroup(k, carry):
            for half in range(2):
                base = (2 * k + half) * NB

                @pl.when(k >= 1)
                def _():
                    for b in range(NB):
                        pltpu.make_async_copy(
                            rows_v.at[half, b], acc_sh.at[didx_v.at[0]],
                            semsc[half]).wait()

                gds = [
                    pltpu.async_copy(
                        t_sh.at[sidx_v.at[base + b]], rows_v.at[half, b], semg)
                    for b in range(NB)
                ]
                for d in gds:
                    d.wait()
                for b in range(NB):
                    pltpu.async_copy(
                        rows_v.at[half, b], acc_sh.at[didx_v.at[base + b]],
                        semsc[half], add=True)
            return carry

        lax.fori_loop(0, n_super, supergroup, 0)

        @pl.when(n_super > 0)
        def _():
            for half in range(2):
                for b in range(NB):
                    pltpu.make_async_copy(
                        rows_v.at[half, b], acc_sh.at[didx_v.at[0]],
                        semsc[half]).wait()

        def tailchunk(i, carry):
            j = n_super * (2 * NB) + i
            pltpu.async_copy(
                t_sh.at[sidx_v.at[j]], rows_v.at[0, 0], semg).wait()
            pltpu.sync_copy(
                rows_v.at[0, 0], acc_sh.at[didx_v.at[j]], add=True)
            return carry

        lax.fori_loop(0, n_tail, tailchunk, 0)
        plsc.subcore_barrier()
        pltpu.sync_copy(acc_sh.at[pl.ds(s * RPT, RPT)],
                        out_hbm.at[c, pl.ds(s * RPT, RPT)])

    return agg_kernel


_BM = 1000


def _tca_body(x_ref, w1_ref, degp_ref, t1_ref, dinv_ref):
    y1 = jnp.dot(x_ref[...], w1_ref[...], preferred_element_type=jnp.float32)
    cnt = degp_ref[0] + degp_ref[1]
    dinv = lax.rsqrt(cnt[:, 0:1] + 1.0)
    t1_ref[...] = y1 * dinv
    dinv_ref[...] = dinv


def _tcb_body(a1p_ref, t1_ref, dinv_ref, b1_ref, wc_ref, tc2_ref):
    agg = a1p_ref[0] + a1p_ref[1] + t1_ref[...]
    h1 = jnp.maximum(agg * dinv_ref[...] + b1_ref[...], 0.0)
    yc = jnp.dot(h1, wc_ref[...], preferred_element_type=jnp.float32)
    tc2_ref[...] = yc * dinv_ref[...]


def _tcc_body(a2p_ref, tc2_ref, dinv_ref, bc_ref, gn_ref, z_ref):
    u = (a2p_ref[0] + a2p_ref[1] + tc2_ref[...]) * dinv_ref[...] + bc_ref[...]
    xu = u[:, :F_LAT]
    xs = u[:, F_LAT:]
    z_ref[...] = gn_ref[...] * jnp.exp(xu) + xs


_DM = 400


def _tcd_body(z1_ref, z2_ref, out_ref):
    zz = lax.dot_general(z1_ref[...], z2_ref[...],
                         (((1,), (1,)), ((), ())),
                         preferred_element_type=jnp.float32)
    out_ref[...] = jax.nn.sigmoid(zz)


def kernel(x, edge_index, W1, b1, W2, b2, W3, b3):
    n_edges = edge_index.shape[1]
    tot = -(-n_edges // CH)
    cm = -(-tot // NW)
    t_rows = tot + (NW * cm - tot > 0) * 1 + 1
    e_pad = t_rows * CH

    ei = edge_index.astype(jnp.int32)
    ei3 = jnp.concatenate(
        [ei, jnp.full((2, e_pad - n_edges), N, jnp.int32)],
        axis=1).reshape(2, t_rows, CH)

    degp = _make_deg_kernel(tot)(ei3)

    xp = jnp.concatenate(
        [x, jnp.zeros((N_PAD - N, F_IN), jnp.float32)], axis=0)
    grid_p = (N_PAD // RPT,)
    t1, dinv = pl.pallas_call(
        _tca_body,
        grid=grid_p,
        in_specs=[
            pl.BlockSpec((RPT, F_IN), lambda i: (i, 0)),
            pl.BlockSpec((F_IN, F_HID), lambda i: (0, 0)),
            pl.BlockSpec((NC, RPT, 16), lambda i: (0, i, 0)),
        ],
        out_specs=[
            pl.BlockSpec((RPT, F_HID), lambda i: (i, 0)),
            pl.BlockSpec((RPT, 1), lambda i: (i, 0)),
        ],
        out_shape=[
            jax.ShapeDtypeStruct((N_PAD, F_HID), jnp.float32),
            jax.ShapeDtypeStruct((N_PAD, 1), jnp.float32),
        ],
    )(xp, W1, degp)

    a1p = _make_agg_kernel(tot, F_HID, 1)(ei3, t1)

    Wc = jnp.concatenate([W2, W3], axis=1)
    bc = jnp.concatenate([b2, b3]).reshape(1, F_C)
    tc2 = pl.pallas_call(
        _tcb_body,
        grid=grid_p,
        in_specs=[
            pl.BlockSpec((NC, RPT, F_HID), lambda i: (0, i, 0)),
            pl.BlockSpec((RPT, F_HID), lambda i: (i, 0)),
            pl.BlockSpec((RPT, 1), lambda i: (i, 0)),
            pl.BlockSpec((1, F_HID), lambda i: (0, 0)),
            pl.BlockSpec((F_HID, F_C), lambda i: (0, 0)),
        ],
        out_specs=pl.BlockSpec((RPT, F_C), lambda i: (i, 0)),
        out_shape=jax.ShapeDtypeStruct((N_PAD, F_C), jnp.float32),
    )(a1p, t1, dinv, b1.reshape(1, F_HID), Wc)

    a2p = _make_agg_kernel(tot, F_C, 2)(ei3, tc2)

    gnoise = jax.random.normal(jax.random.key(42), (N, F_LAT), dtype=jnp.float32)
    grid = (N // _BM,)
    z = pl.pallas_call(
        _tcc_body,
        grid=grid,
        in_specs=[
            pl.BlockSpec((NC, _BM, F_C), lambda i: (0, i, 0)),
            pl.BlockSpec((_BM, F_C), lambda i: (i, 0)),
            pl.BlockSpec((_BM, 1), lambda i: (i, 0)),
            pl.BlockSpec((1, F_C), lambda i: (0, 0)),
            pl.BlockSpec((_BM, F_LAT), lambda i: (i, 0)),
        ],
        out_specs=pl.BlockSpec((_BM, F_LAT), lambda i: (i, 0)),
        out_shape=jax.ShapeDtypeStruct((N, F_LAT), jnp.float32),
    )(a2p, tc2, dinv, bc, gnoise)

    adj = pl.pallas_call(
        _tcd_body,
        grid=(N // _DM,),
        in_specs=[
            pl.BlockSpec((_DM, F_LAT), lambda i: (i, 0)),
            pl.BlockSpec((N, F_LAT), lambda i: (0, 0)),
        ],
        out_specs=pl.BlockSpec((_DM, N), lambda i: (i, 0)),
        out_shape=jax.ShapeDtypeStruct((N, N), jnp.float32),
    )(z, z)
    return adj


# speedup vs baseline: 2.0935x; 1.0422x over previous
import functools

import jax
import jax.numpy as jnp
from jax import lax
from jax.experimental import pallas as pl
from jax.experimental.pallas import tpu as pltpu
from jax.experimental.pallas import tpu_sc as plsc

N = 10000
F_IN = 128
F_HID = 64
F_LAT = 16
F_C = 2 * F_LAT

NC = 2
NS = 16
NW = NC * NS
CH = 128

N_PAD = 10112
RPT = N_PAD // NS

_MESH = plsc.VectorSubcoreMesh(core_axis_name="c", subcore_axis_name="s")


def _zero_fill(buf, rows, width):
    zv = jnp.zeros((16,), jnp.float32)

    def body(r, carry):
        for j in range(width // 16):
            buf[r, pl.ds(j * 16, 16)] = zv
        return carry

    lax.fori_loop(0, rows, body, 0)


NB_MAX = 4


def _tile_plan(tot, c, s):
    wid = (s * NC + c).astype(jnp.int32)
    base = tot // NW
    extra = tot % NW
    cnt = jnp.where(wid < extra, base + 1, base)
    start = wid * base + jnp.minimum(wid, extra)
    return cnt, start


def _make_deg_kernel(tot):
    cm = -(-tot // NW)

    @functools.partial(
        pl.kernel,
        mesh=_MESH,
        out_type=jax.ShapeDtypeStruct((NC, N_PAD, 16), jnp.float32),
        compiler_params=pltpu.CompilerParams(use_tc_tiling_on_sc=False),
        scratch_types=[
            pltpu.VMEM((cm, CH), jnp.int32),
            pltpu.VMEM((CH, 16), jnp.float32),
            pltpu.VMEM((RPT, 16), jnp.float32),
            pltpu.VMEM_SHARED((N_PAD, 16), jnp.float32),
            pltpu.SemaphoreType.DMA,
        ],
    )
    def deg_kernel(ei_hbm, out_hbm, idx_v, ones_v, zbuf, acc_sh, sem):
        c = lax.axis_index("c")
        s = lax.axis_index("s")
        cnt, start = _tile_plan(tot, c, s)

        one = jnp.ones((16,), jnp.float32)

        def fill_ones(r, carry):
            ones_v[r, :] = one
            return carry

        lax.fori_loop(0, CH, fill_ones, 0)
        _zero_fill(zbuf, RPT, 16)
        pltpu.sync_copy(zbuf, acc_sh.at[pl.ds(s * RPT, RPT)])
        pltpu.sync_copy(ei_hbm.at[1, pl.ds(start, cm)], idx_v)
        plsc.subcore_barrier()

        def chunk(i, carry):
            pltpu.async_copy(ones_v, acc_sh.at[idx_v.at[i]], sem, add=True)
            return carry

        lax.fori_loop(0, cnt, chunk, 0)

        def drain(i, carry):
            pltpu.make_async_copy(ones_v, acc_sh.at[idx_v.at[0]], sem).wait()
            return carry

        lax.fori_loop(0, cnt, drain, 0)
        plsc.subcore_barrier()
        pltpu.sync_copy(acc_sh.at[pl.ds(s * RPT, RPT)],
                        out_hbm.at[c, pl.ds(s * RPT, RPT)])

    return deg_kernel


def _make_agg_kernel(tot, feat, NB):
    cm = -(-tot // NW)

    @functools.partial(
        pl.kernel,
        mesh=_MESH,
        out_type=jax.ShapeDtypeStruct((NC, N_PAD, feat), jnp.float32),
        compiler_params=pltpu.CompilerParams(use_tc_tiling_on_sc=False),
        scratch_types=[
            pltpu.VMEM((cm, CH), jnp.int32),
            pltpu.VMEM((cm, CH), jnp.int32),
            pltpu.VMEM((2, NB, CH, feat), jnp.float32),
            pltpu.VMEM((RPT // 4, feat), jnp.float32),
            pltpu.VMEM_SHARED((N_PAD, feat), jnp.float32),
            pltpu.VMEM_SHARED((N_PAD, feat), jnp.float32),
            pltpu.SemaphoreType.DMA,
            pltpu.SemaphoreType.DMA,
            pltpu.SemaphoreType.DMA,
        ],
    )
    def agg_kernel(ei_hbm, t_hbm, out_hbm,
                   sidx_v, didx_v, rows_v, zbuf, acc_sh, t_sh,
                   semg, semsc0, semsc1):
        c = lax.axis_index("c")
        s = lax.axis_index("s")
        cnt, start = _tile_plan(tot, c, s)
        n_super = cnt // (2 * NB)
        n_tail = cnt - n_super * (2 * NB)
        semsc = (semsc0, semsc1)

        _zero_fill(zbuf, RPT // 4, feat)
        for q in range(4):
            pltpu.sync_copy(
                zbuf, acc_sh.at[pl.ds(s * RPT + q * (RPT // 4), RPT // 4)])
        pltpu.sync_copy(t_hbm.at[pl.ds(s * RPT, RPT)],
                        t_sh.at[pl.ds(s * RPT, RPT)])
        pltpu.sync_copy(ei_hbm.at[0, pl.ds(start, cm)], sidx_v)
        pltpu.sync_copy(ei_hbm.at[1, pl.ds(start, cm)], didx_v)
        plsc.subcore_barrier()

        def supergroup(k, carry):
            for half in range(2):
                base = (2 * k + half) * NB

                @pl.when(k >= 1)
                def _():
                    for b in range(NB):
                        pltpu.make_async_copy(
                            rows_v.at[half, b], acc_sh.at[didx_v.at[0]],
                            semsc[half]).wait()

                gds = [
                    pltpu.async_copy(
                        t_sh.at[sidx_v.at[base + b]], rows_v.at[half, b], semg)
                    for b in range(NB)
                ]
                for d in gds:
                    d.wait()
                for b in range(NB):
                    pltpu.async_copy(
                        rows_v.at[half, b], acc_sh.at[didx_v.at[base + b]],
                        semsc[half], add=True)
            return carry

        lax.fori_loop(0, n_super, supergroup, 0)

        @pl.when(n_super > 0)
        def _():
            for half in range(2):
                for b in range(NB):
                    pltpu.make_async_copy(
                        rows_v.at[half, b], acc_sh.at[didx_v.at[0]],
                        semsc[half]).wait()

        def tailchunk(i, carry):
            j = n_super * (2 * NB) + i
            pltpu.async_copy(
                t_sh.at[sidx_v.at[j]], rows_v.at[0, 0], semg).wait()
            pltpu.sync_copy(
                rows_v.at[0, 0], acc_sh.at[didx_v.at[j]], add=True)
            return carry

        lax.fori_loop(0, n_tail, tailchunk, 0)
        plsc.subcore_barrier()
        pltpu.sync_copy(acc_sh.at[pl.ds(s * RPT, RPT)],
                        out_hbm.at[c, pl.ds(s * RPT, RPT)])

    return agg_kernel


_BM = 1000


def _tca_body(x_ref, w1_ref, degp_ref, t1_ref, dinv_ref):
    y1 = jnp.dot(x_ref[...], w1_ref[...], preferred_element_type=jnp.float32)
    cnt = degp_ref[0] + degp_ref[1]
    dinv = lax.rsqrt(cnt[:, 0:1] + 1.0)
    t1_ref[...] = y1 * dinv
    dinv_ref[...] = dinv


def _tcb_body(a1p_ref, t1_ref, dinv_ref, b1_ref, wc_ref, tc2_ref):
    agg = a1p_ref[0] + a1p_ref[1] + t1_ref[...]
    h1 = jnp.maximum(agg * dinv_ref[...] + b1_ref[...], 0.0)
    yc = jnp.dot(h1, wc_ref[...], preferred_element_type=jnp.float32)
    tc2_ref[...] = yc * dinv_ref[...]


def _tcc_body(a2p_ref, tc2_ref, dinv_ref, bc_ref, gn_ref, z_ref):
    u = (a2p_ref[0] + a2p_ref[1] + tc2_ref[...]) * dinv_ref[...] + bc_ref[...]
    xu = u[:, :F_LAT]
    xs = u[:, F_LAT:]
    z_ref[...] = gn_ref[...] * jnp.exp(xu) + xs


_DM = 400


def _tcd_body(z1_ref, z2_ref, out_ref):
    zz = lax.dot_general(z1_ref[...], z2_ref[...],
                         (((1,), (1,)), ((), ())),
                         preferred_element_type=jnp.float32)
    out_ref[...] = jax.nn.sigmoid(zz)


def kernel(x, edge_index, W1, b1, W2, b2, W3, b3):
    n_edges = edge_index.shape[1]
    tot = -(-n_edges // CH)
    cm = -(-tot // NW)
    t_rows = tot + (NW * cm - tot > 0) * 1 + 1
    e_pad = t_rows * CH

    ei = edge_index.astype(jnp.int32)
    ei3 = jnp.concatenate(
        [ei, jnp.full((2, e_pad - n_edges), N, jnp.int32)],
        axis=1).reshape(2, t_rows, CH)

    degp = _make_deg_kernel(tot)(ei3)

    xp = jnp.concatenate(
        [x, jnp.zeros((N_PAD - N, F_IN), jnp.float32)], axis=0)
    t1, dinv = pl.pallas_call(
        _tca_body,
        out_shape=[
            jax.ShapeDtypeStruct((N_PAD, F_HID), jnp.float32),
            jax.ShapeDtypeStruct((N_PAD, 1), jnp.float32),
        ],
    )(xp, W1, degp)

    a1p = _make_agg_kernel(tot, F_HID, 1)(ei3, t1)

    Wc = jnp.concatenate([W2, W3], axis=1)
    bc = jnp.concatenate([b2, b3]).reshape(1, F_C)
    tc2 = pl.pallas_call(
        _tcb_body,
        out_shape=jax.ShapeDtypeStruct((N_PAD, F_C), jnp.float32),
    )(a1p, t1, dinv, b1.reshape(1, F_HID), Wc)

    a2p = _make_agg_kernel(tot, F_C, 4)(ei3, tc2)

    gnoise = jax.random.normal(jax.random.key(42), (N, F_LAT), dtype=jnp.float32)
    z = pl.pallas_call(
        _tcc_body,
        grid=(1,),
        in_specs=[
            pl.BlockSpec((NC, N, F_C), lambda i: (0, 0, 0)),
            pl.BlockSpec((N, F_C), lambda i: (0, 0)),
            pl.BlockSpec((N, 1), lambda i: (0, 0)),
            pl.BlockSpec((1, F_C), lambda i: (0, 0)),
            pl.BlockSpec((N, F_LAT), lambda i: (0, 0)),
        ],
        out_specs=pl.BlockSpec((N, F_LAT), lambda i: (0, 0)),
        out_shape=jax.ShapeDtypeStruct((N, F_LAT), jnp.float32),
    )(a2p, tc2, dinv, bc, gnoise)

    adj = pl.pallas_call(
        _tcd_body,
        grid=(N // _DM,),
        in_specs=[
            pl.BlockSpec((_DM, F_LAT), lambda i: (i, 0)),
            pl.BlockSpec((N, F_LAT), lambda i: (0, 0)),
        ],
        out_specs=pl.BlockSpec((_DM, N), lambda i: (i, 0)),
        out_shape=jax.ShapeDtypeStruct((N, N), jnp.float32),
    )(z, z)
    return adj
